# baseline (speedup 1.0000x reference)
import functools

import numpy as np

import jax
import jax.numpy as jnp
from jax import lax
from jax.experimental import pallas as pl
from jax.experimental.pallas import tpu as pltpu
from jax.experimental.pallas import tpu_sc as plsc

N = 100000
E = 1600000
EMB = 32
HID = 64

L = 16
NC = 2
NS = 16
NW = NC * NS

SUB = 128
SUBC = 4
CH = SUB * SUBC
NCH = 98
NIT = NCH // 2
EPAD = NW * CH * NCH
IDXROWS = EPAD // SUB
WROWS = IDXROWS // NW

NPAD = 100352
RPT = NPAD // NS

N8 = NPAD // 8
NROWS = N // 8
F1 = 8 * EMB
F2 = 8 * HID
FBLK = 448
GRID = N8 // FBLK


def _sc_segment_sums(sd, tables, with_deg):
    S = len(tables)
    mesh = plsc.VectorSubcoreMesh(core_axis_name="c", subcore_axis_name="s")
    out_type = []
    if with_deg:
        out_type.append(jax.ShapeDtypeStruct((NC, NPAD, L), jnp.float32))
    out_type.append(jax.ShapeDtypeStruct((NC, S, NPAD, L), jnp.float32))

    def body(*refs):
        sd_h, zeros_h = refs[0], refs[1]
        tbls = refs[2:2 + S]
        pos = 2 + S
        deg_out = None
        if with_deg:
            deg_out = refs[pos]
            pos += 1
        sum_out = refs[pos]
        nscr = 13 if with_deg else 12
        scr = refs[pos + 1:pos + 1 + nscr]
        acc, sdA, sdB, rowsA, rowsB = scr[:5]
        if with_deg:
            ones_v = scr[5]
            semIA, semIB, semGA, semGB, semSA, semSB, semZ = scr[6:]
        else:
            semIA, semIB, semGA, semGB, semSA, semSB, semZ = scr[5:]

        c = lax.axis_index("c")
        t = lax.axis_index("s")
        wid = t * NC + c
        wbase = wid * WROWS

        if with_deg:
            def _init_o(i, carry):
                ones_v[i] = jnp.ones((L,), jnp.float32)
                return carry
            lax.fori_loop(0, SUB, _init_o, 0)

        def _zero_acc():
            pltpu.sync_copy(zeros_h.at[pl.ds(t * RPT, RPT)],
                            acc.at[pl.ds(t * RPT, RPT)])

        def _fire_idx(ci, buf, sem):
            pltpu.async_copy(sd_h.at[pl.ds(wbase + ci * SUBC, SUBC)], buf,
                             sem)

        def _drain_idx(ci, buf, sem):
            pltpu.make_async_copy(sd_h.at[pl.ds(wbase + ci * SUBC, SUBC)],
                                  buf, sem).wait()

        def _fire_scat(srcbuf, sd, sem, replicate_src):
            for j in range(SUBC):
                s_ref = srcbuf if replicate_src else srcbuf.at[
                    pl.ds(j * SUB, SUB)]
                pltpu.async_copy(s_ref, acc.at[sd.at[j, 1]], sem, add=True)

        def _drain_scat(srcbuf, sd, sem, replicate_src):
            for j in range(SUBC):
                s_ref = srcbuf if replicate_src else srcbuf.at[
                    pl.ds(j * SUB, SUB)]
                pltpu.make_async_copy(s_ref, acc.at[sd.at[j, 1]], sem).wait()

        if with_deg:
            _zero_acc()
            _fire_idx(0, sdA, semIA)
            plsc.subcore_barrier()

            def _deg_it(k, carry):
                b = 2 * k + 1
                _drain_idx(2 * k, sdA, semIA)

                @pl.when(k > 0)
                def _():
                    _drain_scat(ones_v, sdB, semSB, True)
                _fire_idx(b, sdB, semIB)
                _fire_scat(ones_v, sdA, semSA, True)
                _drain_idx(b, sdB, semIB)
                _drain_scat(ones_v, sdA, semSA, True)

                @pl.when(k < NIT - 1)
                def _():
                    _fire_idx(2 * k + 2, sdA, semIA)
                _fire_scat(ones_v, sdB, semSB, True)
                return carry
            lax.fori_loop(0, NIT, _deg_it, 0)
            _drain_scat(ones_v, sdB, semSB, True)
            plsc.subcore_barrier()
            pltpu.sync_copy(acc.at[pl.ds(t * RPT, RPT)],
                            deg_out.at[c, pl.ds(t * RPT, RPT)])
            plsc.subcore_barrier()

        for si in range(S):
            tbl = tbls[si]
            _zero_acc()
            _fire_idx(0, sdA, semIA)
            plsc.subcore_barrier()

            def _fire_gath(sd, rows, sem, _tbl=tbl):
                for j in range(SUBC):
                    pltpu.async_copy(_tbl.at[sd.at[j, 0]],
                                     rows.at[pl.ds(j * SUB, SUB)], sem)

            def _drain_gath(sd, rows, sem, _tbl=tbl):
                for j in range(SUBC):
                    pltpu.make_async_copy(_tbl.at[sd.at[j, 0]],
                                          rows.at[pl.ds(j * SUB, SUB)],
                                          sem).wait()

            def _it(k, carry, _fg=_fire_gath, _dg=_drain_gath):
                b = 2 * k + 1
                _drain_idx(2 * k, sdA, semIA)
                _fg(sdA, rowsA, semGA)

                @pl.when(k > 0)
                def _():
                    _drain_scat(rowsB, sdB, semSB, False)
                _fire_idx(b, sdB, semIB)
                _dg(sdA, rowsA, semGA)
                _fire_scat(rowsA, sdA, semSA, False)
                _drain_idx(b, sdB, semIB)
                _fg(sdB, rowsB, semGB)
                _drain_scat(rowsA, sdA, semSA, False)

                @pl.when(k < NIT - 1)
                def _():
                    _fire_idx(2 * k + 2, sdA, semIA)
                _dg(sdB, rowsB, semGB)
                _fire_scat(rowsB, sdB, semSB, False)
                return carry
            lax.fori_loop(0, NIT, _it, 0)
            _drain_scat(rowsB, sdB, semSB, False)
            plsc.subcore_barrier()
            pltpu.sync_copy(acc.at[pl.ds(t * RPT, RPT)],
                            sum_out.at[c, si, pl.ds(t * RPT, RPT)])
            plsc.subcore_barrier()

    scratch = [
        pltpu.VMEM_SHARED((NPAD, L), jnp.float32),
        pltpu.VMEM((SUBC, 2, SUB), jnp.int32),
        pltpu.VMEM((SUBC, 2, SUB), jnp.int32),
        pltpu.VMEM((CH, L), jnp.float32),
        pltpu.VMEM((CH, L), jnp.float32),
    ]
    if with_deg:
        scratch.append(pltpu.VMEM((SUB, L), jnp.float32))
    scratch += [pltpu.SemaphoreType.DMA] * 7
    f = pl.kernel(
        body,
        out_type=tuple(out_type),
        mesh=mesh,
        scratch_types=scratch,
        compiler_params=pltpu.CompilerParams(use_tc_tiling_on_sc=False),
    )
    zeros_h = jnp.zeros((NPAD, L), jnp.float32)
    return f(sd, zeros_h, *tables)


def _tc_layer1(embf, deg_pf, sum_pf, bd_self, bd_n0, bd_n1, b1t, Pj):
    def body(embf_b, dp_b, sp_b, ws_b, wn0_b, wn1_b, bt_b, P_b, h1s_o,
             degf_o):
        deg = jnp.maximum(dp_b[0] + dp_b[1], 1.0)
        degf_o[...] = deg
        m0 = (sp_b[0, 0] + sp_b[1, 0]) / deg
        m1 = (sp_b[0, 1] + sp_b[1, 1]) / deg
        h = (jnp.dot(embf_b[...], ws_b[...], preferred_element_type=jnp.float32)
             + jnp.dot(m0, wn0_b[...], preferred_element_type=jnp.float32)
             + jnp.dot(m1, wn1_b[...], preferred_element_type=jnp.float32)
             + bt_b[...])
        h1 = jnp.maximum(h, 0.0)
        for s in range(4):
            h1s_o[s] = jnp.dot(h1, P_b[s], preferred_element_type=jnp.float32)

    return pl.pallas_call(
        body,
        grid=(GRID,),
        in_specs=[
            pl.BlockSpec((FBLK, F1), lambda i: (i, 0)),
            pl.BlockSpec((NC, FBLK, 128), lambda i: (0, i, 0)),
            pl.BlockSpec((NC, 2, FBLK, 128), lambda i: (0, 0, i, 0)),
            pl.BlockSpec((F1, F2), lambda i: (0, 0)),
            pl.BlockSpec((128, F2), lambda i: (0, 0)),
            pl.BlockSpec((128, F2), lambda i: (0, 0)),
            pl.BlockSpec((1, F2), lambda i: (0, 0)),
            pl.BlockSpec((4, F2, 128), lambda i: (0, 0, 0)),
        ],
        out_specs=[
            pl.BlockSpec((4, FBLK, 128), lambda i: (0, i, 0)),
            pl.BlockSpec((FBLK, 128), lambda i: (i, 0)),
        ],
        out_shape=[
            jax.ShapeDtypeStruct((4, N8, 128), jnp.float32),
            jax.ShapeDtypeStruct((N8, 128), jnp.float32),
        ],
    )(embf, deg_pf, sum_pf, bd_self, bd_n0, bd_n1, b1t, Pj)


def _tc_layer2(h1s4, sum_pf, degf, bd_s2, bd_ns, b2t, Rp, Ws1p, bs1p, ws2p,
               bs2p):
    def body(h1s_b, sp_b, dg_b, ws0_b, ws1s_b, ws2s_b, ws3_b, wn0_b, wn1_b,
             wn2_b, wn3_b, bt_b, R_b, ws1_b, bs1_b, ws2_b, bs2_b, out_o,
             accv):
        i = pl.program_id(0)
        deg = dg_b[...]
        wss = [ws0_b, ws1s_b, ws2s_b, ws3_b]
        wns = [wn0_b, wn1_b, wn2_b, wn3_b]
        h = bt_b[...]
        for si in range(4):
            h = h + jnp.dot(h1s_b[si], wss[si][...],
                            preferred_element_type=jnp.float32)
            m = (sp_b[0, si] + sp_b[1, si]) / deg
            h = h + jnp.dot(m, wns[si][...],
                            preferred_element_type=jnp.float32)
        h2 = jnp.maximum(h, 0.0)
        rix = lax.broadcasted_iota(jnp.int32, (FBLK, F2), 0) + i * FBLK
        h2 = jnp.where(rix < NROWS, h2, 0.0)
        part = jnp.sum(h2, axis=0, keepdims=True)

        @pl.when(i == 0)
        def _():
            accv[...] = part

        @pl.when(i > 0)
        def _():
            accv[...] = accv[...] + part

        @pl.when(i == GRID - 1)
        def _():
            hg = jnp.dot(accv[...], R_b[...],
                         preferred_element_type=jnp.float32) / jnp.float32(N)
            sv = jnp.maximum(
                jnp.dot(hg, ws1_b[...], preferred_element_type=jnp.float32)
                + bs1_b[...], 0.0)
            scal = jnp.sum(sv * ws2_b[...])
            out_o[...] = jnp.full((1, 128), scal, jnp.float32) + bs2_b[...]

    return pl.pallas_call(
        body,
        grid=(GRID,),
        in_specs=[
            pl.BlockSpec((4, FBLK, 128), lambda i: (0, i, 0)),
            pl.BlockSpec((NC, 4, FBLK, 128), lambda i: (0, 0, i, 0)),
            pl.BlockSpec((FBLK, 128), lambda i: (i, 0)),
            pl.BlockSpec((128, F2), lambda i: (0, 0)),
            pl.BlockSpec((128, F2), lambda i: (0, 0)),
            pl.BlockSpec((128, F2), lambda i: (0, 0)),
            pl.BlockSpec((128, F2), lambda i: (0, 0)),
            pl.BlockSpec((128, F2), lambda i: (0, 0)),
            pl.BlockSpec((128, F2), lambda i: (0, 0)),
            pl.BlockSpec((128, F2), lambda i: (0, 0)),
            pl.BlockSpec((128, F2), lambda i: (0, 0)),
            pl.BlockSpec((1, F2), lambda i: (0, 0)),
            pl.BlockSpec((F2, 128), lambda i: (0, 0)),
            pl.BlockSpec((128, 128), lambda i: (0, 0)),
            pl.BlockSpec((1, 128), lambda i: (0, 0)),
            pl.BlockSpec((1, 128), lambda i: (0, 0)),
            pl.BlockSpec((1, 128), lambda i: (0, 0)),
        ],
        out_specs=pl.BlockSpec((1, 128), lambda i: (0, 0)),
        out_shape=jax.ShapeDtypeStruct((1, 128), jnp.float32),
        scratch_shapes=[pltpu.VMEM((1, F2), jnp.float32)],
    )(h1s4, sum_pf, degf, bd_s2[0], bd_s2[1], bd_s2[2], bd_s2[3],
      bd_ns[0], bd_ns[1], bd_ns[2], bd_ns[3],
      b2t, Rp, Ws1p, bs1p, ws2p, bs2p)


def kernel(node_ids, edge_index, emb, W_self1, W_neigh1, b1, W_self2,
           W_neigh2, b2, Ws1, bs1, Ws2, bs2):
    f32 = jnp.float32
    src = edge_index[0]
    dst = edge_index[1]
    pad = EPAD - E
    srcp = jnp.concatenate([src, jnp.zeros((pad,), jnp.int32)]).reshape(
        IDXROWS, SUB)
    dstp = jnp.concatenate([dst, jnp.full((pad,), N, jnp.int32)]).reshape(
        IDXROWS, SUB)
    sd = jnp.stack([srcp, dstp], axis=1)
    e0 = emb[:, :L]
    e1 = emb[:, L:]
    deg_p, sum1_p = _sc_segment_sums(sd, [e0, e1], with_deg=True)

    embf = emb.reshape(NROWS, F1)
    deg_pf = deg_p.reshape(NC, N8, 128)
    sum1_pf = sum1_p.reshape(NC, 2, N8, 128)
    eye8 = jnp.eye(8, dtype=f32)
    bd_self1 = jnp.kron(eye8, W_self1)
    bd_n1 = [jnp.kron(eye8, W_neigh1[s * L:(s + 1) * L, :]) for s in range(2)]
    b1t = jnp.tile(b1, 8).reshape(1, F2)
    P_np = np.zeros((4, F2, 128), np.float32)
    for s in range(4):
        for cp in range(128):
            P_np[s, 64 * (cp // 16) + 16 * s + (cp % 16), cp] = 1.0
    Pj = jnp.asarray(P_np)
    h1s4, degf = _tc_layer1(embf, deg_pf, sum1_pf, bd_self1, bd_n1[0],
                            bd_n1[1], b1t, Pj)

    h1t = h1s4.reshape(4, NPAD, L)
    h1s = [h1t[s] for s in range(4)]
    sum2_p = _sc_segment_sums(sd, h1s, with_deg=False)
    if isinstance(sum2_p, (list, tuple)):
        (sum2_p,) = sum2_p
    sum2_pf = sum2_p.reshape(NC, 4, N8, 128)

    bd_s2 = [jnp.kron(eye8, W_self2[s * L:(s + 1) * L, :]) for s in range(4)]
    bd_n2 = [jnp.kron(eye8, W_neigh2[s * L:(s + 1) * L, :]) for s in range(4)]
    b2t = jnp.tile(b2, 8).reshape(1, F2)
    Rp = jnp.zeros((F2, 128), f32).at[:, :HID].set(
        jnp.tile(jnp.eye(HID, dtype=f32), (8, 1)))
    Ws1p = jnp.zeros((128, 128), f32).at[:HID, :HID].set(Ws1)
    bs1p = jnp.zeros((1, 128), f32).at[0, :HID].set(bs1)
    ws2p = jnp.zeros((1, 128), f32).at[0, :HID].set(Ws2[:, 0])
    bs2p = jnp.zeros((1, 128), f32).at[0, 0].set(bs2[0])
    outv = _tc_layer2(h1s4, sum2_pf, degf, bd_s2, bd_n2, b2t, Rp, Ws1p,
                      bs1p, ws2p, bs2p)
    return outv[0, :1]


# speedup vs baseline: 1.3083x; 1.3083x over previous
import functools

import numpy as np

import jax
import jax.numpy as jnp
from jax import lax
from jax.experimental import pallas as pl
from jax.experimental.pallas import tpu as pltpu
from jax.experimental.pallas import tpu_sc as plsc

N = 100000
E = 1600000
EMB = 32
HID = 64

L = 16
NC = 2
NS = 16
NW = NC * NS

SUB = 128
SUBC = 4
CH = SUB * SUBC
NCH = 98
NIT = NCH // 2
EPAD = NW * CH * NCH
IDXROWS = EPAD // SUB
WROWS = IDXROWS // NW

NPAD = 100352
RPT = NPAD // NS

N8 = NPAD // 8
NROWS = N // 8
F1 = 8 * EMB
F2 = 8 * HID
FBLK = 448
GRID = N8 // FBLK


def _sc_segment_sums(sd, tables, with_deg, feat_w=16, feat_dtype=jnp.float32):
    S = len(tables)
    mesh = plsc.VectorSubcoreMesh(core_axis_name="c", subcore_axis_name="s")
    out_type = []
    if with_deg:
        out_type.append(jax.ShapeDtypeStruct((NC, NPAD, L), jnp.float32))
    out_type.append(jax.ShapeDtypeStruct((NC, S, NPAD, feat_w), feat_dtype))

    def body(*refs):
        sd_h, zeros_h = refs[0], refs[1]
        tbls = refs[2:2 + S]
        pos = 2 + S
        deg_out = None
        if with_deg:
            deg_out = refs[pos]
            pos += 1
        sum_out = refs[pos]
        nscr = 13 if with_deg else 12
        scr = refs[pos + 1:pos + 1 + nscr]
        acc, sdA, sdB, rowsA, rowsB = scr[:5]
        if with_deg:
            ones_v = scr[5]
            semIA, semIB, semGA, semGB, semSA, semSB, semZ = scr[6:]
        else:
            semIA, semIB, semGA, semGB, semSA, semSB, semZ = scr[5:]

        c = lax.axis_index("c")
        t = lax.axis_index("s")
        wid = t * NC + c
        wbase = wid * WROWS

        if with_deg:
            def _init_o(i, carry):
                ones_v[i] = jnp.ones((L,), jnp.float32)
                return carry
            lax.fori_loop(0, SUB, _init_o, 0)

        def _zero_acc():
            pltpu.sync_copy(zeros_h.at[pl.ds(t * RPT, RPT)],
                            acc.at[pl.ds(t * RPT, RPT)])

        def _fire_idx(ci, buf, sem):
            pltpu.async_copy(sd_h.at[pl.ds(wbase + ci * SUBC, SUBC)], buf,
                             sem)

        def _drain_idx(ci, buf, sem):
            pltpu.make_async_copy(sd_h.at[pl.ds(wbase + ci * SUBC, SUBC)],
                                  buf, sem).wait()

        def _fire_scat(srcbuf, sd, sem, replicate_src):
            for j in range(SUBC):
                s_ref = srcbuf if replicate_src else srcbuf.at[
                    pl.ds(j * SUB, SUB)]
                pltpu.async_copy(s_ref, acc.at[sd.at[j, 1]], sem, add=True)

        def _drain_scat(srcbuf, sd, sem, replicate_src):
            for j in range(SUBC):
                s_ref = srcbuf if replicate_src else srcbuf.at[
                    pl.ds(j * SUB, SUB)]
                pltpu.make_async_copy(s_ref, acc.at[sd.at[j, 1]], sem).wait()

        if with_deg:
            _zero_acc()
            _fire_idx(0, sdA, semIA)
            plsc.subcore_barrier()

            def _deg_it(k, carry):
                b = 2 * k + 1
                _drain_idx(2 * k, sdA, semIA)

                @pl.when(k > 0)
                def _():
                    _drain_scat(ones_v, sdB, semSB, True)
                _fire_idx(b, sdB, semIB)
                _fire_scat(ones_v, sdA, semSA, True)
                _drain_idx(b, sdB, semIB)
                _drain_scat(ones_v, sdA, semSA, True)

                @pl.when(k < NIT - 1)
                def _():
                    _fire_idx(2 * k + 2, sdA, semIA)
                _fire_scat(ones_v, sdB, semSB, True)
                return carry
            lax.fori_loop(0, NIT, _deg_it, 0)
            _drain_scat(ones_v, sdB, semSB, True)
            plsc.subcore_barrier()
            pltpu.sync_copy(acc.at[pl.ds(t * RPT, RPT)],
                            deg_out.at[c, pl.ds(t * RPT, RPT)])
            plsc.subcore_barrier()

        for si in range(S):
            tbl = tbls[si]
            _zero_acc()
            _fire_idx(0, sdA, semIA)
            plsc.subcore_barrier()

            def _fire_gath(sd, rows, sem, _tbl=tbl):
                for j in range(SUBC):
                    pltpu.async_copy(_tbl.at[sd.at[j, 0]],
                                     rows.at[pl.ds(j * SUB, SUB)], sem)

            def _drain_gath(sd, rows, sem, _tbl=tbl):
                for j in range(SUBC):
                    pltpu.make_async_copy(_tbl.at[sd.at[j, 0]],
                                          rows.at[pl.ds(j * SUB, SUB)],
                                          sem).wait()

            def _it(k, carry, _fg=_fire_gath, _dg=_drain_gath):
                b = 2 * k + 1
                _drain_idx(2 * k, sdA, semIA)
                _fg(sdA, rowsA, semGA)

                @pl.when(k > 0)
                def _():
                    _drain_scat(rowsB, sdB, semSB, False)
                _fire_idx(b, sdB, semIB)
                _dg(sdA, rowsA, semGA)
                _fire_scat(rowsA, sdA, semSA, False)
                _drain_idx(b, sdB, semIB)
                _fg(sdB, rowsB, semGB)
                _drain_scat(rowsA, sdA, semSA, False)

                @pl.when(k < NIT - 1)
                def _():
                    _fire_idx(2 * k + 2, sdA, semIA)
                _dg(sdB, rowsB, semGB)
                _fire_scat(rowsB, sdB, semSB, False)
                return carry
            lax.fori_loop(0, NIT, _it, 0)
            _drain_scat(rowsB, sdB, semSB, False)
            plsc.subcore_barrier()
            pltpu.sync_copy(acc.at[pl.ds(t * RPT, RPT)],
                            sum_out.at[c, si, pl.ds(t * RPT, RPT)])
            plsc.subcore_barrier()

    scratch = [
        pltpu.VMEM_SHARED((NPAD, feat_w), feat_dtype),
        pltpu.VMEM((SUBC, 2, SUB), jnp.int32),
        pltpu.VMEM((SUBC, 2, SUB), jnp.int32),
        pltpu.VMEM((CH, feat_w), feat_dtype),
        pltpu.VMEM((CH, feat_w), feat_dtype),
    ]
    if with_deg:
        scratch.append(pltpu.VMEM((SUB, L), jnp.float32))
    scratch += [pltpu.SemaphoreType.DMA] * 7
    f = pl.kernel(
        body,
        out_type=tuple(out_type),
        mesh=mesh,
        scratch_types=scratch,
        compiler_params=pltpu.CompilerParams(use_tc_tiling_on_sc=False),
    )
    zeros_h = jnp.zeros((NPAD, feat_w), feat_dtype)
    return f(sd, zeros_h, *tables)


def _tc_layer1(embf, deg_pf, sum_pf, bd_self, bd_n0, bd_n1, b1t, Pj, Dj):
    def body(embf_b, dp_b, sp_b, ws_b, wn0_b, wn1_b, bt_b, P_b, D_b, h1f_o,
             t2_o, deg32_o):
        deg = jnp.maximum(dp_b[0] + dp_b[1], 1.0)
        deg32_o[...] = jnp.dot(deg, D_b[...],
                               preferred_element_type=jnp.float32)
        m0 = (sp_b[0, 0] + sp_b[1, 0]) / deg
        m1 = (sp_b[0, 1] + sp_b[1, 1]) / deg
        h = (jnp.dot(embf_b[...], ws_b[...], preferred_element_type=jnp.float32)
             + jnp.dot(m0, wn0_b[...], preferred_element_type=jnp.float32)
             + jnp.dot(m1, wn1_b[...], preferred_element_type=jnp.float32)
             + bt_b[...])
        h1 = jnp.maximum(h, 0.0)
        h1f_o[...] = h1
        for s in range(2):
            t2_o[s] = jnp.dot(h1, P_b[s],
                              preferred_element_type=jnp.float32).astype(
                                  jnp.bfloat16)

    return pl.pallas_call(
        body,
        grid=(GRID,),
        in_specs=[
            pl.BlockSpec((FBLK, F1), lambda i: (i, 0)),
            pl.BlockSpec((NC, FBLK, 128), lambda i: (0, i, 0)),
            pl.BlockSpec((NC, 2, FBLK, 128), lambda i: (0, 0, i, 0)),
            pl.BlockSpec((F1, F2), lambda i: (0, 0)),
            pl.BlockSpec((128, F2), lambda i: (0, 0)),
            pl.BlockSpec((128, F2), lambda i: (0, 0)),
            pl.BlockSpec((1, F2), lambda i: (0, 0)),
            pl.BlockSpec((2, F2, 256), lambda i: (0, 0, 0)),
            pl.BlockSpec((128, 256), lambda i: (0, 0)),
        ],
        out_specs=[
            pl.BlockSpec((FBLK, F2), lambda i: (i, 0)),
            pl.BlockSpec((2, FBLK, 256), lambda i: (0, i, 0)),
            pl.BlockSpec((FBLK, 256), lambda i: (i, 0)),
        ],
        out_shape=[
            jax.ShapeDtypeStruct((N8, F2), jnp.float32),
            jax.ShapeDtypeStruct((2, N8, 256), jnp.bfloat16),
            jax.ShapeDtypeStruct((N8, 256), jnp.float32),
        ],
    )(embf, deg_pf, sum_pf, bd_self, bd_n0, bd_n1, b1t, Pj, Dj)


def _tc_layer2(h1f, sum_pf, deg32, bd_self, bd_ns, b2t, Rp, Ws1p, bs1p, ws2p,
               bs2p):
    def body(h1f_b, sp_b, dg_b, ws_b, wn0_b, wn1_b, bt_b, R_b,
             ws1_b, bs1_b, ws2_b, bs2_b, out_o, accv):
        i = pl.program_id(0)
        deg = dg_b[...]
        wns = [wn0_b, wn1_b]
        h = bt_b[...] + jnp.dot(h1f_b[...], ws_b[...],
                                preferred_element_type=jnp.float32)
        for si in range(2):
            m = (sp_b[0, si] + sp_b[1, si]).astype(jnp.float32) / deg
            h = h + jnp.dot(m, wns[si][...],
                            preferred_element_type=jnp.float32)
        h2 = jnp.maximum(h, 0.0)
        rix = lax.broadcasted_iota(jnp.int32, (FBLK, F2), 0) + i * FBLK
        h2 = jnp.where(rix < NROWS, h2, 0.0)
        part = jnp.sum(h2, axis=0, keepdims=True)

        @pl.when(i == 0)
        def _():
            accv[...] = part

        @pl.when(i > 0)
        def _():
            accv[...] = accv[...] + part

        @pl.when(i == GRID - 1)
        def _():
            hg = jnp.dot(accv[...], R_b[...],
                         preferred_element_type=jnp.float32) / jnp.float32(N)
            sv = jnp.maximum(
                jnp.dot(hg, ws1_b[...], preferred_element_type=jnp.float32)
                + bs1_b[...], 0.0)
            scal = jnp.sum(sv * ws2_b[...])
            out_o[...] = jnp.full((1, 128), scal, jnp.float32) + bs2_b[...]

    return pl.pallas_call(
        body,
        grid=(GRID,),
        in_specs=[
            pl.BlockSpec((FBLK, F2), lambda i: (i, 0)),
            pl.BlockSpec((NC, 2, FBLK, 256), lambda i: (0, 0, i, 0)),
            pl.BlockSpec((FBLK, 256), lambda i: (i, 0)),
            pl.BlockSpec((F2, F2), lambda i: (0, 0)),
            pl.BlockSpec((256, F2), lambda i: (0, 0)),
            pl.BlockSpec((256, F2), lambda i: (0, 0)),
            pl.BlockSpec((1, F2), lambda i: (0, 0)),
            pl.BlockSpec((F2, 128), lambda i: (0, 0)),
            pl.BlockSpec((128, 128), lambda i: (0, 0)),
            pl.BlockSpec((1, 128), lambda i: (0, 0)),
            pl.BlockSpec((1, 128), lambda i: (0, 0)),
            pl.BlockSpec((1, 128), lambda i: (0, 0)),
        ],
        out_specs=pl.BlockSpec((1, 128), lambda i: (0, 0)),
        out_shape=jax.ShapeDtypeStruct((1, 128), jnp.float32),
        scratch_shapes=[pltpu.VMEM((1, F2), jnp.float32)],
    )(h1f, sum_pf, deg32, bd_self, bd_ns[0], bd_ns[1],
      b2t, Rp, Ws1p, bs1p, ws2p, bs2p)


def kernel(node_ids, edge_index, emb, W_self1, W_neigh1, b1, W_self2,
           W_neigh2, b2, Ws1, bs1, Ws2, bs2):
    f32 = jnp.float32
    src = edge_index[0]
    dst = edge_index[1]
    pad = EPAD - E
    srcp = jnp.concatenate([src, jnp.zeros((pad,), jnp.int32)]).reshape(
        IDXROWS, SUB)
    dstp = jnp.concatenate([dst, jnp.full((pad,), N, jnp.int32)]).reshape(
        IDXROWS, SUB)
    sd = jnp.stack([srcp, dstp], axis=1)
    e0 = emb[:, :L]
    e1 = emb[:, L:]
    deg_p, sum1_p = _sc_segment_sums(sd, [e0, e1], with_deg=True)

    embp = jnp.concatenate([emb, jnp.zeros((NPAD - N, EMB), f32)])
    embf = embp.reshape(N8, F1)
    deg_pf = deg_p.reshape(NC, N8, 128)
    sum1_pf = sum1_p.reshape(NC, 2, N8, 128)
    eye8 = jnp.eye(8, dtype=f32)
    bd_self1 = jnp.kron(eye8, W_self1)
    bd_n1 = [jnp.kron(eye8, W_neigh1[s * L:(s + 1) * L, :]) for s in range(2)]
    b1t = jnp.tile(b1, 8).reshape(1, F2)
    P_np = np.zeros((2, F2, 256), np.float32)
    for s in range(2):
        for cp in range(256):
            P_np[s, 64 * (cp // 32) + 32 * s + (cp % 32), cp] = 1.0
    Pj = jnp.asarray(P_np)
    D_np = np.zeros((128, 256), np.float32)
    for cp in range(256):
        D_np[16 * (cp // 32), cp] = 1.0
    Dj = jnp.asarray(D_np)
    h1f, t2, deg32 = _tc_layer1(embf, deg_pf, sum1_pf, bd_self1, bd_n1[0],
                                bd_n1[1], b1t, Pj, Dj)

    t2r = t2.reshape(2, NPAD, 2 * L)
    h1s = [t2r[s] for s in range(2)]
    sum2_p = _sc_segment_sums(sd, h1s, with_deg=False, feat_w=2 * L,
                              feat_dtype=jnp.bfloat16)
    if isinstance(sum2_p, (list, tuple)):
        (sum2_p,) = sum2_p
    sum2_pf = sum2_p.reshape(NC, 2, N8, 256)

    bd_self2 = jnp.kron(eye8, W_self2)
    bd_n2 = [jnp.kron(eye8, W_neigh2[s * 2 * L:(s + 1) * 2 * L, :])
             for s in range(2)]
    b2t = jnp.tile(b2, 8).reshape(1, F2)
    Rp = jnp.zeros((F2, 128), f32).at[:, :HID].set(
        jnp.tile(jnp.eye(HID, dtype=f32), (8, 1)))
    Ws1p = jnp.zeros((128, 128), f32).at[:HID, :HID].set(Ws1)
    bs1p = jnp.zeros((1, 128), f32).at[0, :HID].set(bs1)
    ws2p = jnp.zeros((1, 128), f32).at[0, :HID].set(Ws2[:, 0])
    bs2p = jnp.zeros((1, 128), f32).at[0, 0].set(bs2[0])
    outv = _tc_layer2(h1f, sum2_pf, deg32, bd_self2, bd_n2, b2t, Rp, Ws1p,
                      bs1p, ws2p, bs2p)
    return outv[0, :1]


# speedup vs baseline: 1.4721x; 1.1252x over previous
import functools

import numpy as np

import jax
import jax.numpy as jnp
from jax import lax
from jax.experimental import pallas as pl
from jax.experimental.pallas import tpu as pltpu
from jax.experimental.pallas import tpu_sc as plsc

N = 100000
E = 1600000
EMB = 32
HID = 64

L = 16
NC = 2
NS = 16
NW = NC * NS

SUB = 128
SUBC = 4
CH = SUB * SUBC
NCH = 98
NIT = NCH // 2
EPAD = NW * CH * NCH
IDXROWS = EPAD // SUB
WROWS = IDXROWS // NW

NPAD = 100352
RPT = NPAD // NS

N8 = NPAD // 8
NROWS = N // 8
F1 = 8 * EMB
F2 = 8 * HID
FBLK = 448
GRID = N8 // FBLK


def _sc_segment_sums(sd, tables, with_deg, feat_w=16, feat_dtype=jnp.float32):
    S = len(tables)
    mesh = plsc.VectorSubcoreMesh(core_axis_name="c", subcore_axis_name="s")
    out_type = []
    if with_deg:
        out_type.append(jax.ShapeDtypeStruct((NC, NPAD, feat_w), feat_dtype))
    out_type.append(jax.ShapeDtypeStruct((NC, S, NPAD, feat_w), feat_dtype))

    def body(*refs):
        sd_h, zeros_h = refs[0], refs[1]
        tbls = refs[2:2 + S]
        pos = 2 + S
        deg_out = None
        if with_deg:
            deg_out = refs[pos]
            pos += 1
        sum_out = refs[pos]
        nscr = 13 if with_deg else 12
        scr = refs[pos + 1:pos + 1 + nscr]
        acc, sdA, sdB, rowsA, rowsB = scr[:5]
        if with_deg:
            ones_v = scr[5]
            semIA, semIB, semGA, semGB, semSA, semSB, semZ = scr[6:]
        else:
            semIA, semIB, semGA, semGB, semSA, semSB, semZ = scr[5:]

        c = lax.axis_index("c")
        t = lax.axis_index("s")
        wid = t * NC + c
        wbase = wid * WROWS

        if with_deg:
            def _init_o(i, carry):
                ones_v[i] = jnp.ones((feat_w,), feat_dtype)
                return carry
            lax.fori_loop(0, SUB, _init_o, 0)

        def _zero_acc():
            pltpu.sync_copy(zeros_h.at[pl.ds(t * RPT, RPT)],
                            acc.at[pl.ds(t * RPT, RPT)])

        def _fire_idx(ci, buf, sem):
            pltpu.async_copy(sd_h.at[pl.ds(wbase + ci * SUBC, SUBC)], buf,
                             sem)

        def _drain_idx(ci, buf, sem):
            pltpu.make_async_copy(sd_h.at[pl.ds(wbase + ci * SUBC, SUBC)],
                                  buf, sem).wait()

        def _fire_scat(srcbuf, sd, sem, replicate_src):
            for j in range(SUBC):
                s_ref = srcbuf if replicate_src else srcbuf.at[
                    pl.ds(j * SUB, SUB)]
                pltpu.async_copy(s_ref, acc.at[sd.at[j, 1]], sem, add=True)

        def _drain_scat(srcbuf, sd, sem, replicate_src):
            for j in range(SUBC):
                s_ref = srcbuf if replicate_src else srcbuf.at[
                    pl.ds(j * SUB, SUB)]
                pltpu.make_async_copy(s_ref, acc.at[sd.at[j, 1]], sem).wait()

        if with_deg:
            _zero_acc()
            _fire_idx(0, sdA, semIA)
            plsc.subcore_barrier()

            def _deg_it(k, carry):
                b = 2 * k + 1
                _drain_idx(2 * k, sdA, semIA)

                @pl.when(k > 0)
                def _():
                    _drain_scat(ones_v, sdB, semSB, True)
                _fire_idx(b, sdB, semIB)
                _fire_scat(ones_v, sdA, semSA, True)
                _drain_idx(b, sdB, semIB)
                _drain_scat(ones_v, sdA, semSA, True)

                @pl.when(k < NIT - 1)
                def _():
                    _fire_idx(2 * k + 2, sdA, semIA)
                _fire_scat(ones_v, sdB, semSB, True)
                return carry
            lax.fori_loop(0, NIT, _deg_it, 0)
            _drain_scat(ones_v, sdB, semSB, True)
            plsc.subcore_barrier()
            pltpu.sync_copy(acc.at[pl.ds(t * RPT, RPT)],
                            deg_out.at[c, pl.ds(t * RPT, RPT)])
            plsc.subcore_barrier()

        for si in range(S):
            tbl = tbls[si]
            _zero_acc()
            _fire_idx(0, sdA, semIA)
            plsc.subcore_barrier()

            def _fire_gath(sd, rows, sem, _tbl=tbl):
                for j in range(SUBC):
                    pltpu.async_copy(_tbl.at[sd.at[j, 0]],
                                     rows.at[pl.ds(j * SUB, SUB)], sem)

            def _drain_gath(sd, rows, sem, _tbl=tbl):
                for j in range(SUBC):
                    pltpu.make_async_copy(_tbl.at[sd.at[j, 0]],
                                          rows.at[pl.ds(j * SUB, SUB)],
                                          sem).wait()

            def _it(k, carry, _fg=_fire_gath, _dg=_drain_gath):
                b = 2 * k + 1
                _drain_idx(2 * k, sdA, semIA)
                _fg(sdA, rowsA, semGA)

                @pl.when(k > 0)
                def _():
                    _drain_scat(rowsB, sdB, semSB, False)
                _fire_idx(b, sdB, semIB)
                _dg(sdA, rowsA, semGA)
                _fire_scat(rowsA, sdA, semSA, False)
                _drain_idx(b, sdB, semIB)
                _fg(sdB, rowsB, semGB)
                _drain_scat(rowsA, sdA, semSA, False)

                @pl.when(k < NIT - 1)
                def _():
                    _fire_idx(2 * k + 2, sdA, semIA)
                _dg(sdB, rowsB, semGB)
                _fire_scat(rowsB, sdB, semSB, False)
                return carry
            lax.fori_loop(0, NIT, _it, 0)
            _drain_scat(rowsB, sdB, semSB, False)
            plsc.subcore_barrier()
            pltpu.sync_copy(acc.at[pl.ds(t * RPT, RPT)],
                            sum_out.at[c, si, pl.ds(t * RPT, RPT)])
            plsc.subcore_barrier()

    scratch = [
        pltpu.VMEM_SHARED((NPAD, feat_w), feat_dtype),
        pltpu.VMEM((SUBC, 2, SUB), jnp.int32),
        pltpu.VMEM((SUBC, 2, SUB), jnp.int32),
        pltpu.VMEM((CH, feat_w), feat_dtype),
        pltpu.VMEM((CH, feat_w), feat_dtype),
    ]
    if with_deg:
        scratch.append(pltpu.VMEM((SUB, feat_w), feat_dtype))
    scratch += [pltpu.SemaphoreType.DMA] * 7
    f = pl.kernel(
        body,
        out_type=tuple(out_type),
        mesh=mesh,
        scratch_types=scratch,
        compiler_params=pltpu.CompilerParams(use_tc_tiling_on_sc=False),
    )
    zeros_h = jnp.zeros((NPAD, feat_w), feat_dtype)
    return f(sd, zeros_h, *tables)


def _tc_layer1(embf, deg_pf, sum_pf, bd_self, bd_n, b1t, Pj):
    def body(embf_b, dp_b, sp_b, ws_b, wn_b, bt_b, P_b, h1f_o, t2_o,
             deg32_o):
        deg = jnp.maximum(dp_b[0].astype(jnp.float32)
                          + dp_b[1].astype(jnp.float32), 1.0)
        deg32_o[...] = deg
        m = (sp_b[0].astype(jnp.float32) + sp_b[1].astype(jnp.float32)) / deg
        h = (jnp.dot(embf_b[...], ws_b[...], preferred_element_type=jnp.float32)
             + jnp.dot(m, wn_b[...], preferred_element_type=jnp.float32)
             + bt_b[...])
        h1 = jnp.maximum(h, 0.0)
        h1f_o[...] = h1
        for s in range(2):
            t2_o[s] = jnp.dot(h1, P_b[s],
                              preferred_element_type=jnp.float32).astype(
                                  jnp.bfloat16)

    return pl.pallas_call(
        body,
        grid=(GRID,),
        in_specs=[
            pl.BlockSpec((FBLK, F1), lambda i: (i, 0)),
            pl.BlockSpec((NC, FBLK, 256), lambda i: (0, i, 0)),
            pl.BlockSpec((NC, FBLK, 256), lambda i: (0, i, 0)),
            pl.BlockSpec((F1, F2), lambda i: (0, 0)),
            pl.BlockSpec((256, F2), lambda i: (0, 0)),
            pl.BlockSpec((1, F2), lambda i: (0, 0)),
            pl.BlockSpec((2, F2, 256), lambda i: (0, 0, 0)),
        ],
        out_specs=[
            pl.BlockSpec((FBLK, F2), lambda i: (i, 0)),
            pl.BlockSpec((2, FBLK, 256), lambda i: (0, i, 0)),
            pl.BlockSpec((FBLK, 256), lambda i: (i, 0)),
        ],
        out_shape=[
            jax.ShapeDtypeStruct((N8, F2), jnp.float32),
            jax.ShapeDtypeStruct((2, N8, 256), jnp.bfloat16),
            jax.ShapeDtypeStruct((N8, 256), jnp.float32),
        ],
    )(embf, deg_pf, sum_pf, bd_self, bd_n, b1t, Pj)


def _tc_layer2(h1f, sum_pf, deg32, bd_self, bd_ns, b2t, Rp, Ws1p, bs1p, ws2p,
               bs2p):
    def body(h1f_b, sp_b, dg_b, ws_b, wn0_b, wn1_b, bt_b, R_b,
             ws1_b, bs1_b, ws2_b, bs2_b, out_o, accv):
        i = pl.program_id(0)
        deg = dg_b[...]
        wns = [wn0_b, wn1_b]
        h = bt_b[...] + jnp.dot(h1f_b[...], ws_b[...],
                                preferred_element_type=jnp.float32)
        for si in range(2):
            m = (sp_b[0, si] + sp_b[1, si]).astype(jnp.float32) / deg
            h = h + jnp.dot(m, wns[si][...],
                            preferred_element_type=jnp.float32)
        h2 = jnp.maximum(h, 0.0)
        rix = lax.broadcasted_iota(jnp.int32, (FBLK, F2), 0) + i * FBLK
        h2 = jnp.where(rix < NROWS, h2, 0.0)
        part = jnp.sum(h2, axis=0, keepdims=True)

        @pl.when(i == 0)
        def _():
            accv[...] = part

        @pl.when(i > 0)
        def _():
            accv[...] = accv[...] + part

        @pl.when(i == GRID - 1)
        def _():
            hg = jnp.dot(accv[...], R_b[...],
                         preferred_element_type=jnp.float32) / jnp.float32(N)
            sv = jnp.maximum(
                jnp.dot(hg, ws1_b[...], preferred_element_type=jnp.float32)
                + bs1_b[...], 0.0)
            scal = jnp.sum(sv * ws2_b[...])
            out_o[...] = jnp.full((1, 128), scal, jnp.float32) + bs2_b[...]

    return pl.pallas_call(
        body,
        grid=(GRID,),
        in_specs=[
            pl.BlockSpec((FBLK, F2), lambda i: (i, 0)),
            pl.BlockSpec((NC, 2, FBLK, 256), lambda i: (0, 0, i, 0)),
            pl.BlockSpec((FBLK, 256), lambda i: (i, 0)),
            pl.BlockSpec((F2, F2), lambda i: (0, 0)),
            pl.BlockSpec((256, F2), lambda i: (0, 0)),
            pl.BlockSpec((256, F2), lambda i: (0, 0)),
            pl.BlockSpec((1, F2), lambda i: (0, 0)),
            pl.BlockSpec((F2, 128), lambda i: (0, 0)),
            pl.BlockSpec((128, 128), lambda i: (0, 0)),
            pl.BlockSpec((1, 128), lambda i: (0, 0)),
            pl.BlockSpec((1, 128), lambda i: (0, 0)),
            pl.BlockSpec((1, 128), lambda i: (0, 0)),
        ],
        out_specs=pl.BlockSpec((1, 128), lambda i: (0, 0)),
        out_shape=jax.ShapeDtypeStruct((1, 128), jnp.float32),
        scratch_shapes=[pltpu.VMEM((1, F2), jnp.float32)],
    )(h1f, sum_pf, deg32, bd_self, bd_ns[0], bd_ns[1],
      b2t, Rp, Ws1p, bs1p, ws2p, bs2p)


def kernel(node_ids, edge_index, emb, W_self1, W_neigh1, b1, W_self2,
           W_neigh2, b2, Ws1, bs1, Ws2, bs2):
    f32 = jnp.float32
    src = edge_index[0]
    dst = edge_index[1]
    pad = EPAD - E
    srcp = jnp.concatenate([src, jnp.zeros((pad,), jnp.int32)]).reshape(
        IDXROWS, SUB)
    dstp = jnp.concatenate([dst, jnp.full((pad,), N, jnp.int32)]).reshape(
        IDXROWS, SUB)
    sd = jnp.stack([srcp, dstp], axis=1)
    e32 = emb.astype(jnp.bfloat16)
    deg_p, sum1_p = _sc_segment_sums(sd, [e32], with_deg=True, feat_w=2 * L,
                                     feat_dtype=jnp.bfloat16)

    embp = jnp.concatenate([emb, jnp.zeros((NPAD - N, EMB), f32)])
    embf = embp.reshape(N8, F1)
    deg_pf = deg_p.reshape(NC, N8, 256)
    sum1_pf = sum1_p.reshape(NC, N8, 256)
    eye8 = jnp.eye(8, dtype=f32)
    bd_self1 = jnp.kron(eye8, W_self1)
    bd_n1 = jnp.kron(eye8, W_neigh1)
    b1t = jnp.tile(b1, 8).reshape(1, F2)
    P_np = np.zeros((2, F2, 256), np.float32)
    for s in range(2):
        for cp in range(256):
            P_np[s, 64 * (cp // 32) + 32 * s + (cp % 32), cp] = 1.0
    Pj = jnp.asarray(P_np)
    h1f, t2, deg32 = _tc_layer1(embf, deg_pf, sum1_pf, bd_self1, bd_n1, b1t,
                                Pj)

    t2r = t2.reshape(2, NPAD, 2 * L)
    h1s = [t2r[s] for s in range(2)]
    sum2_p = _sc_segment_sums(sd, h1s, with_deg=False, feat_w=2 * L,
                              feat_dtype=jnp.bfloat16)
    if isinstance(sum2_p, (list, tuple)):
        (sum2_p,) = sum2_p
    sum2_pf = sum2_p.reshape(NC, 2, N8, 256)

    bd_self2 = jnp.kron(eye8, W_self2)
    bd_n2 = [jnp.kron(eye8, W_neigh2[s * 2 * L:(s + 1) * 2 * L, :])
             for s in range(2)]
    b2t = jnp.tile(b2, 8).reshape(1, F2)
    Rp = jnp.zeros((F2, 128), f32).at[:, :HID].set(
        jnp.tile(jnp.eye(HID, dtype=f32), (8, 1)))
    Ws1p = jnp.zeros((128, 128), f32).at[:HID, :HID].set(Ws1)
    bs1p = jnp.zeros((1, 128), f32).at[0, :HID].set(bs1)
    ws2p = jnp.zeros((1, 128), f32).at[0, :HID].set(Ws2[:, 0])
    bs2p = jnp.zeros((1, 128), f32).at[0, 0].set(bs2[0])
    outv = _tc_layer2(h1f, sum2_pf, deg32, bd_self2, bd_n2, b2t, Rp, Ws1p,
                      bs1p, ws2p, bs2p)
    return outv[0, :1]
